# baseline (speedup 1.0000x reference)
import functools

import jax
import jax.numpy as jnp
from jax import lax
from jax.experimental import pallas as pl
from jax.experimental.pallas import tpu as pltpu
from jax.experimental.pallas import tpu_sc as plsc

_INFO = plsc.get_sparse_core_info()
_NC, _NS = _INFO.num_cores, _INFO.num_subcores
_NW = _NC * _NS
_L = 16

_VBLK = 512


def _pack_kernel(emb_t_ref, out_ref):
    t = emb_t_ref[...].T
    t3 = t.reshape(_VBLK // 2, 2, 64)
    out_ref[:, 0:64] = t3[:, 0, :]
    out_ref[:, 64:128] = t3[:, 1, :]


@functools.lru_cache(maxsize=None)
def _make_pack(V, D):
    grid = (V + _VBLK - 1) // _VBLK
    return pl.pallas_call(
        _pack_kernel,
        grid=(grid,),
        in_specs=[pl.BlockSpec((D, _VBLK), lambda i: (0, i))],
        out_specs=pl.BlockSpec((_VBLK // 2, 2 * D), lambda i: (i, 0)),
        out_shape=jax.ShapeDtypeStruct((V // 2, 2 * D), jnp.float32),
    )


@functools.lru_cache(maxsize=None)
def _make_gather(B, H, D):
    b_per_w = B // _NW
    CH = 256
    n_sub = b_per_w // CH
    mesh = plsc.VectorSubcoreMesh(core_axis_name="c", subcore_axis_name="s")

    scratch = (
        [pltpu.VMEM((CH,), jnp.int32) for _ in range(2)]
        + [pltpu.VMEM((CH,), jnp.int32) for _ in range(2)]
        + [pltpu.VMEM((CH, 2 * D), jnp.float32) for _ in range(2)]
        + [pltpu.VMEM((D, 128), jnp.float32) for _ in range(2)]
        + [
            pltpu.SemaphoreType.DMA,
            pltpu.SemaphoreType.DMA,
            pltpu.SemaphoreType.DMA,
        ]
    )

    @functools.partial(
        pl.kernel,
        mesh=mesh,
        out_type=jax.ShapeDtypeStruct((H, D, B), jnp.float32),
        scratch_types=scratch,
        compiler_params=pltpu.CompilerParams(
            use_tc_tiling_on_sc=True, needs_layout_passes=False
        ),
    )
    def gather_kernel(xt_hbm, table_hbm, out_hbm, *bufs):
        idx_v = bufs[0:2]
        half_v = bufs[2:4]
        rows_v = bufs[4:6]
        tiles_v = bufs[6:8]
        sem_i, sem_g, sem_o = bufs[8], bufs[9], bufs[10]

        wid = lax.axis_index("s") * _NC + lax.axis_index("c")
        b0 = wid * b_per_w
        lane = lax.iota(jnp.int32, _L)
        total = H * n_sub

        def idx_src(step):
            h = step // n_sub
            sub = lax.rem(step, n_sub)
            return xt_hbm.at[h, pl.ds(b0 + sub * CH, CH)]

        def start_idx(step, s):
            pltpu.async_copy(idx_src(step), idx_v[s], sem_i)

        def wait_idx(step, s):
            pltpu.make_async_copy(idx_src(step), idx_v[s], sem_i).wait()

        def halve(s):
            for g in range(CH // _L):
                v = idx_v[s][pl.ds(g * _L, _L)]
                half_v[s][pl.ds(g * _L, _L)] = lax.shift_right_logical(v, 1)

        def start_gather(s):
            pltpu.async_copy(table_hbm.at[half_v[s]], rows_v[s], sem_g)

        def wait_gather(s):
            pltpu.make_async_copy(table_hbm.at[half_v[s]], rows_v[s], sem_g).wait()

        start_idx(0, 0)
        wait_idx(0, 0)
        halve(0)
        start_gather(0)
        start_idx(1, 1)

        def process(step, s):
            h = step // n_sub
            sub = lax.rem(step, n_sub)
            for blk in range(CH // 128):
                t = blk
                @pl.when(step > 0)
                def _(t=t):
                    for dj in range(D // 8):
                        pltpu.make_async_copy(
                            tiles_v[t].at[pl.ds(dj * 8, 8), pl.ds(0, 128)],
                            out_hbm.at[0, pl.ds(dj * 8, 8), pl.ds(0, 128)],
                            sem_o,
                        ).wait()

                bs0 = blk * 128
                rvecs, qvecs = [], []
                for g in range(128 // _L):
                    vraw = idx_v[s][pl.ds(bs0 + g * _L, _L)]
                    q = lax.bitwise_and(vraw, 1)
                    rvecs.append(bs0 + g * _L + lane)
                    qvecs.append(q * 64)

                def dbody(d, carry):
                    for g in range(128 // _L):
                        vals = plsc.load_gather(
                            rows_v[s], [rvecs[g], qvecs[g] + d]
                        )
                        tiles_v[t][d, pl.ds(g * _L, _L)] = vals
                    return carry

                lax.fori_loop(0, D, dbody, 0, unroll=2)

                bg = wid * (b_per_w // 128) + sub * (CH // 128) + blk
                for dj in range(D // 8):
                    pltpu.async_copy(
                        tiles_v[t].at[pl.ds(dj * 8, 8), pl.ds(0, 128)],
                        out_hbm.at[h, pl.ds(dj * 8, 8), pl.ds(bg * 128, 128)],
                        sem_o,
                    )

        def body(step, carry):
            s = lax.rem(step, 2)

            @pl.when(s == 0)
            def _():
                wait_gather(0)

                @pl.when(step + 1 < total)
                def _():
                    wait_idx(step + 1, 1)
                    halve(1)
                    start_gather(1)

                process(step, 0)

                @pl.when(step + 2 < total)
                def _():
                    start_idx(step + 2, 0)

            @pl.when(s == 1)
            def _():
                wait_gather(1)

                @pl.when(step + 1 < total)
                def _():
                    wait_idx(step + 1, 0)
                    halve(0)
                    start_gather(0)

                process(step, 1)

                @pl.when(step + 2 < total)
                def _():
                    start_idx(step + 2, 1)

            return carry

        lax.fori_loop(0, total, body, 0)

        for t in range(2):
            for dj in range(D // 8):
                pltpu.make_async_copy(
                    tiles_v[t].at[pl.ds(dj * 8, 8), pl.ds(0, 128)],
                    out_hbm.at[0, pl.ds(dj * 8, 8), pl.ds(0, 128)],
                    sem_o,
                ).wait()

    return gather_kernel


def kernel(x, embedding):
    batch, hist = x.shape
    vocab, dim = embedding.shape
    table = _make_pack(vocab, dim)(embedding.T)
    xt = x.T
    out = _make_gather(batch, hist, dim)(xt, table)
    return out.transpose(2, 0, 1)
